# baseline (speedup 1.0000x reference)
import functools

import jax
import jax.numpy as jnp
from jax import lax
from jax.experimental import pallas as pl
from jax.experimental.pallas import tpu as pltpu
from jax.experimental.pallas import tpu_sc as plsc

_NUM_CORES = 2
_NUM_SUBCORES = 16
_LANES = 16
_IDX_CHUNK = 128


def _build_sc_kernel(q_len, v_len, vocab, dim):
  mp = (vocab - 1) // 2
  n_workers = _NUM_CORES * _NUM_SUBCORES
  assert q_len % n_workers == 0
  rows_per_worker = q_len // n_workers
  slice_rows = v_len + rows_per_worker - 1
  slice_pad = -slice_rows % _IDX_CHUNK
  n_chunks = (slice_rows + slice_pad) // _IDX_CHUNK
  assert dim % _LANES == 0

  mesh = plsc.VectorSubcoreMesh(
      core_axis_name="c", subcore_axis_name="s")

  @functools.partial(
      pl.kernel,
      out_type=jax.ShapeDtypeStruct((q_len, v_len, dim), jnp.float32),
      mesh=mesh,
      scratch_types=[
          pltpu.VMEM((n_chunks, _IDX_CHUNK), jnp.int32),
          pltpu.VMEM((slice_rows + slice_pad, dim), jnp.float32),
          pltpu.SemaphoreType.DMA,
      ],
      compiler_params=pltpu.CompilerParams(use_tc_tiling_on_sc=False),
  )
  def body(weight_hbm, out_hbm, idx_v, l_v, sem):
    c = lax.axis_index("c")
    s = lax.axis_index("s")
    wid = s * _NUM_CORES + c
    base = wid * rows_per_worker

    off0 = mp + 1 - rows_per_worker - base
    lanes = lax.iota(jnp.int32, _LANES)
    for ch in range(n_chunks):
      for k in range(_IDX_CHUNK // _LANES):
        m0 = ch * _IDX_CHUNK + k * _LANES
        vals = jnp.clip(lanes + (m0 + off0), 0, 2 * mp)
        idx_v[ch, pl.ds(k * _LANES, _LANES)] = vals

    copies = [
        pltpu.async_copy(
            weight_hbm.at[idx_v.at[ch]],
            l_v.at[pl.ds(ch * _IDX_CHUNK, _IDX_CHUNK)],
            sem)
        for ch in range(n_chunks)
    ]
    for cp in copies:
      cp.wait()

    row_copies = [
        pltpu.async_copy(
            l_v.at[pl.ds(rows_per_worker - 1 - r, v_len)],
            out_hbm.at[base + r],
            sem)
        for r in range(rows_per_worker)
    ]
    for cp in row_copies:
      cp.wait()

  return body


def kernel(query, value, weight):
  q_len = query.shape[1]
  v_len = value.shape[1]
  vocab, dim = weight.shape
  sc = _build_sc_kernel(q_len, v_len, vocab, dim)
  return sc(weight)


# speedup vs baseline: 1.2379x; 1.2379x over previous
import functools

import jax
import jax.numpy as jnp
from jax import lax
from jax.experimental import pallas as pl
from jax.experimental.pallas import tpu as pltpu
from jax.experimental.pallas import tpu_sc as plsc

_NUM_CORES = 2
_NUM_SUBCORES = 16
_LANES = 16
_CHUNK = 128


def _build_sc_kernel(q_len, v_len, vocab, dim):
  mp = (vocab - 1) // 2
  g_len = q_len + v_len - 1
  g_pad = -g_len % _CHUNK
  g_rows = g_len + g_pad
  n_chunks = g_rows // _CHUNK
  chunks_per_tile = n_chunks // _NUM_SUBCORES
  n_workers = _NUM_CORES * _NUM_SUBCORES
  assert q_len % n_workers == 0
  rows_per_worker = q_len // n_workers
  assert dim % _LANES == 0

  mesh = plsc.VectorSubcoreMesh(
      core_axis_name="c", subcore_axis_name="s")

  @functools.partial(
      pl.kernel,
      out_type=[
          jax.ShapeDtypeStruct((q_len, v_len, dim), jnp.float32),
          jax.ShapeDtypeStruct((g_rows, dim), jnp.float32),
      ],
      mesh=mesh,
      scratch_types=[
          pltpu.VMEM((chunks_per_tile, _CHUNK), jnp.int32),
          pltpu.VMEM((chunks_per_tile * _CHUNK, dim), jnp.float32),
          pltpu.VMEM_SHARED((g_rows, dim), jnp.float32),
          pltpu.SemaphoreType.DMA,
      ],
      compiler_params=pltpu.CompilerParams(use_tc_tiling_on_sc=False),
  )
  def body(weight_hbm, out_hbm, g_hbm, idx_v, buf_v, g, sem):
    c = lax.axis_index("c")
    s = lax.axis_index("s")

    lanes = lax.iota(jnp.int32, _LANES)
    toff = -(q_len - 1 - mp)
    for ci in range(chunks_per_tile):
      chunk = s + ci * _NUM_SUBCORES
      t0 = chunk * _CHUNK
      for k in range(_CHUNK // _LANES):
        vals = jnp.clip(lanes + (t0 + k * _LANES + toff), 0, 2 * mp)
        idx_v[ci, pl.ds(k * _LANES, _LANES)] = vals
    gathers = [
        pltpu.async_copy(
            weight_hbm.at[idx_v.at[ci]],
            buf_v.at[pl.ds(ci * _CHUNK, _CHUNK)],
            sem)
        for ci in range(chunks_per_tile)
    ]
    for cp in gathers:
      cp.wait()
    stages = [
        pltpu.async_copy(
            buf_v.at[pl.ds(ci * _CHUNK, _CHUNK)],
            g_hbm.at[pl.ds((s + ci * _NUM_SUBCORES) * _CHUNK, _CHUNK)],
            sem)
        for ci in range(chunks_per_tile)
    ]
    for cp in stages:
      cp.wait()

    plsc.subcore_barrier()

    @pl.when(s == 0)
    def _pull():
      pltpu.sync_copy(g_hbm, g)

    plsc.subcore_barrier()

    wid = s * _NUM_CORES + c
    base = wid * rows_per_worker
    row_copies = [
        pltpu.async_copy(
            g.at[pl.ds(q_len - 1 - (base + r), v_len)],
            out_hbm.at[base + r],
            sem)
        for r in range(rows_per_worker)
    ]
    for cp in row_copies:
      cp.wait()

  return body


def kernel(query, value, weight):
  q_len = query.shape[1]
  v_len = value.shape[1]
  vocab, dim = weight.shape
  sc = _build_sc_kernel(q_len, v_len, vocab, dim)
  out, _ = sc(weight)
  return out


# speedup vs baseline: 2.2338x; 1.8046x over previous
import functools

import jax
import jax.numpy as jnp
from jax import lax
from jax.experimental import pallas as pl
from jax.experimental.pallas import tpu as pltpu
from jax.experimental.pallas import tpu_sc as plsc

_NUM_CORES = 2
_NUM_SUBCORES = 16
_LANES = 16
_WIDE = 128
_CHUNK = 128
_PHASES = 4


def _build_sc_kernel(q_len, v_len, vocab, dim):
  mp = (vocab - 1) // 2
  group = _WIDE // dim
  assert group * dim == _WIDE and _PHASES == group
  t4_rows = vocab + group - 1
  toff = -(q_len - 1 - mp)
  g_flat = (q_len + v_len - 1) * dim
  rows_per_phase = -(-g_flat // _WIDE)
  assert rows_per_phase % (_NUM_SUBCORES // _PHASES * 2) == 0
  n_workers = _NUM_CORES * _NUM_SUBCORES
  assert q_len % n_workers == 0
  rows_per_worker = q_len // n_workers
  out_mid = v_len * dim // _WIDE
  win_rows = out_mid
  quarters = _NUM_SUBCORES // _PHASES
  chunks_per_tile = rows_per_phase // _CHUNK // quarters

  mesh = plsc.VectorSubcoreMesh(
      core_axis_name="c", subcore_axis_name="s")

  @functools.partial(
      pl.kernel,
      out_type=[
          jax.ShapeDtypeStruct((q_len, out_mid, _WIDE), jnp.float32),
          jax.ShapeDtypeStruct((_PHASES * rows_per_phase, _WIDE),
                               jnp.float32),
      ],
      mesh=mesh,
      scratch_types=[
          pltpu.VMEM((chunks_per_tile, _CHUNK), jnp.int32),
          pltpu.VMEM((chunks_per_tile * _CHUNK, _WIDE), jnp.float32),
          pltpu.VMEM_SHARED((_PHASES * rows_per_phase, _WIDE),
                            jnp.float32),
          pltpu.SemaphoreType.DMA,
      ],
      compiler_params=pltpu.CompilerParams(use_tc_tiling_on_sc=False),
  )
  def body(t4_hbm, out_hbm, stage_hbm, idx_v, buf_v, g_all, sem):
    c = lax.axis_index("c")
    s = lax.axis_index("s")

    phase = s // quarters
    quarter = s % quarters
    k0_tile = quarter * (chunks_per_tile * _CHUNK)
    lanes = lax.iota(jnp.int32, _LANES)
    for ci in range(chunks_per_tile):
      for kk in range(_CHUNK // _LANES):
        k = k0_tile + ci * _CHUNK + kk * _LANES
        vals = jnp.clip((lanes + k) * group + phase + toff,
                        -(group - 1), 2 * mp) + (group - 1)
        idx_v[ci, pl.ds(kk * _LANES, _LANES)] = vals
    gathers = [
        pltpu.async_copy(
            t4_hbm.at[idx_v.at[ci]],
            buf_v.at[pl.ds(ci * _CHUNK, _CHUNK)],
            sem)
        for ci in range(chunks_per_tile)
    ]
    for cp in gathers:
      cp.wait()
    stages = [
        pltpu.async_copy(
            buf_v.at[pl.ds(ci * _CHUNK, _CHUNK)],
            stage_hbm.at[pl.ds(phase * rows_per_phase + k0_tile + ci * _CHUNK,
                               _CHUNK)],
            sem)
        for ci in range(chunks_per_tile)
    ]
    for cp in stages:
      cp.wait()

    plsc.subcore_barrier()

    @pl.when(s == 0)
    def _pull():
      pltpu.sync_copy(stage_hbm, g_all)

    plsc.subcore_barrier()

    wid = s * _NUM_CORES + c
    base = wid * rows_per_worker
    row_copies = []
    for r in range(rows_per_worker):
      i = base + r
      p = (q_len - 1 - r) % _PHASES
      k0 = (q_len - 1 - i - p) // _PHASES
      row_copies.append(
          pltpu.async_copy(
              g_all.at[pl.ds(p * rows_per_phase + k0, win_rows)],
              out_hbm.at[i],
              sem))
    for cp in row_copies:
      cp.wait()

  return body


def kernel(query, value, weight):
  q_len = query.shape[1]
  v_len = value.shape[1]
  vocab, dim = weight.shape
  group = _WIDE // dim
  vidx = jnp.clip(
      jnp.arange(-(group - 1), vocab)[:, None] + jnp.arange(group)[None, :],
      0, vocab - 1)
  t4 = jnp.reshape(weight[vidx], (vocab + group - 1, group * dim))
  sc = _build_sc_kernel(q_len, v_len, vocab, dim)
  out, _ = sc(t4)
  return jnp.reshape(out, (q_len, v_len, dim))
